# Initial kernel scaffold; baseline (speedup 1.0000x reference)
import jax
import jax.numpy as jnp
from jax.experimental import pallas as pl


def kernel(x, pos, edge_index, batch, params):
    raise NotImplementedError("write your pallas kernel here")


# speedup vs baseline: 29.4053x; 29.4053x over previous
import functools

import jax
import jax.numpy as jnp
from jax import lax
from jax.experimental import pallas as pl
from jax.experimental.pallas import tpu as pltpu
from jax.experimental.pallas import tpu_sc as plsc

N = 10000
NP = 10240
E = 160000
EP = 163840
EW = EP // 32
C = 128
D = 128
MSG = 64
H = 4
DH = 16
L = 4
NUM_RADIAL = 50
CUTOFF = 6.0
NUM_GRAPHS = 100
ATOM_TYPES = 5
ACCW = 80

_MESH = plsc.VectorSubcoreMesh(core_axis_name="c", subcore_axis_name="s")
_SC_PARAMS = pltpu.CompilerParams(needs_layout_passes=False,
                                  use_tc_tiling_on_sc=False)


def _h0_body(x_ref, pos_ref, emb_ref, w_ref, b_ref, o_ref):
    xb = x_ref[...]
    oh = (xb == lax.broadcasted_iota(jnp.int32, (1, ATOM_TYPES), 1))
    ex = jnp.dot(oh.astype(jnp.float32), emb_ref[...],
                 preferred_element_type=jnp.float32, precision=lax.Precision.HIGHEST)
    feat = jnp.concatenate([ex, pos_ref[...]], axis=1)
    o_ref[...] = jnp.dot(feat, w_ref[...],
                         preferred_element_type=jnp.float32, precision=lax.Precision.HIGHEST) + b_ref[...]


def _tc_h0(x2, pos3, emb, w, b):
    blk = 2560
    return pl.pallas_call(
        _h0_body,
        grid=(NP // blk,),
        in_specs=[
            pl.BlockSpec((blk, 1), lambda i: (i, 0)),
            pl.BlockSpec((blk, 3), lambda i: (i, 0)),
            pl.BlockSpec((ATOM_TYPES, ATOM_TYPES), lambda i: (0, 0)),
            pl.BlockSpec((ATOM_TYPES + 3, D), lambda i: (0, 0)),
            pl.BlockSpec((1, D), lambda i: (0, 0)),
        ],
        out_specs=pl.BlockSpec((blk, D), lambda i: (i, 0)),
        out_shape=jax.ShapeDtypeStruct((NP, D), jnp.float32),
    )(x2, pos3, emb, w, b)


def _rbfe_body(dsq_ref, we_ref, o_ref):
    d = jnp.sqrt(dsq_ref[...])
    j = lax.broadcasted_iota(jnp.int32, (1, 64), 1).astype(jnp.float32)
    centers = j * (CUTOFF / (NUM_RADIAL - 1))
    gamma = CUTOFF / NUM_RADIAL
    rbf = jnp.exp(-((d - centers) ** 2) * (1.0 / (2.0 * gamma * gamma)))
    for l in range(L):
        o_ref[l, :, :] = jnp.dot(rbf, we_ref[l],
                                 preferred_element_type=jnp.float32, precision=lax.Precision.HIGHEST)


def _tc_rbfe(dsq2, we_all):
    blk = 2048
    return pl.pallas_call(
        _rbfe_body,
        grid=(EP // blk,),
        in_specs=[
            pl.BlockSpec((blk, 1), lambda i: (i, 0)),
            pl.BlockSpec((L, 64, MSG), lambda i: (0, 0, 0)),
        ],
        out_specs=pl.BlockSpec((L, blk, MSG), lambda i: (0, i, 0)),
        out_shape=jax.ShapeDtypeStruct((L, EP, MSG), jnp.float32),
    )(dsq2, we_all)


def _qkv_body(h_ref, wq_ref, wk_ref, wv_ref, q_ref, k_ref, v_ref):
    hb = h_ref[...]
    q_ref[...] = jnp.dot(hb, wq_ref[...], preferred_element_type=jnp.float32, precision=lax.Precision.HIGHEST)
    k_ref[...] = jnp.dot(hb, wk_ref[...], preferred_element_type=jnp.float32, precision=lax.Precision.HIGHEST)
    v_ref[...] = jnp.dot(hb, wv_ref[...], preferred_element_type=jnp.float32, precision=lax.Precision.HIGHEST)


def _tc_qkv(h, wq, wk, wv):
    blk = 2560
    w_spec = pl.BlockSpec((D, MSG), lambda i: (0, 0))
    o_spec = pl.BlockSpec((blk, MSG), lambda i: (i, 0))
    o_shape = jax.ShapeDtypeStruct((NP, MSG), jnp.float32)
    return pl.pallas_call(
        _qkv_body,
        grid=(NP // blk,),
        in_specs=[pl.BlockSpec((blk, D), lambda i: (i, 0)),
                  w_spec, w_spec, w_spec],
        out_specs=[o_spec, o_spec, o_spec],
        out_shape=[o_shape, o_shape, o_shape],
    )(h, wq, wk, wv)


def _post_body(h_ref, acc_ref, wo_ref, bo_ref, g_ref, b_ref,
               f1w_ref, f1b_ref, f2w_ref, f2b_ref, o_ref):
    a = acc_ref[0] + acc_ref[1]
    parts = []
    for hh in range(H):
        num = a[:, hh * DH:(hh + 1) * DH]
        den = a[:, MSG + hh:MSG + hh + 1]
        parts.append(num / (den + 1e-30))
    msg = jnp.concatenate(parts, axis=1)
    u = jnp.dot(msg, wo_ref[...], preferred_element_type=jnp.float32, precision=lax.Precision.HIGHEST)
    x1 = h_ref[...] + jax.nn.gelu(u + bo_ref[...])
    mu = jnp.mean(x1, axis=1, keepdims=True)
    xc = x1 - mu
    var = jnp.mean(xc * xc, axis=1, keepdims=True)
    x1 = xc / jnp.sqrt(var + 1e-5) * g_ref[...] + b_ref[...]
    x1 = x1 + jax.nn.gelu(
        jnp.dot(x1, f1w_ref[...], preferred_element_type=jnp.float32, precision=lax.Precision.HIGHEST)
        + f1b_ref[...])
    x1 = x1 + jax.nn.gelu(
        jnp.dot(x1, f2w_ref[...], preferred_element_type=jnp.float32, precision=lax.Precision.HIGHEST)
        + f2b_ref[...])
    o_ref[...] = x1


def _tc_post(h, acc, wo, bo, ln_g, ln_b, f1w, f1b, f2w, f2b):
    blk = 2560
    vec = lambda: pl.BlockSpec((1, D), lambda i: (0, 0))
    mat = lambda: pl.BlockSpec((D, D), lambda i: (0, 0))
    return pl.pallas_call(
        _post_body,
        grid=(NP // blk,),
        in_specs=[
            pl.BlockSpec((blk, D), lambda i: (i, 0)),
            pl.BlockSpec((2, blk, ACCW), lambda i: (0, i, 0)),
            pl.BlockSpec((MSG, D), lambda i: (0, 0)), vec(),
            vec(), vec(),
            mat(), vec(), mat(), vec(),
        ],
        out_specs=pl.BlockSpec((blk, D), lambda i: (i, 0)),
        out_shape=jax.ShapeDtypeStruct((NP, D), jnp.float32),
    )(h, acc, wo, bo, ln_g, ln_b, f1w, f1b, f2w, f2b)


def _pool_body(h_ref, b_ref, o_ref):
    i = pl.program_id(0)
    oh = (b_ref[...] == lax.broadcasted_iota(jnp.int32, (1, 128), 1))
    part = lax.dot_general(oh.astype(jnp.float32), h_ref[...],
                           (((0,), (0,)), ((), ())),
                           preferred_element_type=jnp.float32, precision=lax.Precision.HIGHEST)

    @pl.when(i == 0)
    def _():
        o_ref[...] = jnp.zeros_like(o_ref)

    o_ref[...] += part


def _tc_pool(h, batch2):
    blk = 2000
    return pl.pallas_call(
        _pool_body,
        grid=(N // blk,),
        in_specs=[
            pl.BlockSpec((blk, D), lambda i: (i, 0)),
            pl.BlockSpec((blk, 1), lambda i: (i, 0)),
        ],
        out_specs=pl.BlockSpec((128, D), lambda i: (0, 0)),
        out_shape=jax.ShapeDtypeStruct((128, D), jnp.float32),
    )(h, batch2)


def _heads_body(g_ref, w1_ref, b1_ref, w2_ref, b2_ref,
                nw1_ref, nb1_ref, nw2_ref, nb2_ref,
                cw1_ref, cb1_ref, cw2_ref, cb2_ref,
                enc_ref, na_ref, cp_ref):
    g = g_ref[...]
    g = jax.nn.gelu(jnp.dot(g, w1_ref[...],
                            preferred_element_type=jnp.float32, precision=lax.Precision.HIGHEST) + b1_ref[...])
    g = jax.nn.gelu(jnp.dot(g, w2_ref[...],
                            preferred_element_type=jnp.float32, precision=lax.Precision.HIGHEST) + b2_ref[...])
    enc_ref[...] = g[:NUM_GRAPHS, :]
    t = jax.nn.gelu(jnp.dot(g, nw1_ref[...],
                            preferred_element_type=jnp.float32, precision=lax.Precision.HIGHEST) + nb1_ref[...])
    na = jnp.dot(t, nw2_ref[...], preferred_element_type=jnp.float32, precision=lax.Precision.HIGHEST) \
        + nb2_ref[...]
    na_ref[...] = na[:NUM_GRAPHS, :]
    u = jax.nn.gelu(jnp.dot(g, cw1_ref[...],
                            preferred_element_type=jnp.float32, precision=lax.Precision.HIGHEST) + cb1_ref[...])
    cp = jnp.dot(u, cw2_ref[...], preferred_element_type=jnp.float32, precision=lax.Precision.HIGHEST) \
        + cb2_ref[...]
    cp_ref[...] = cp[:NUM_GRAPHS, :]


def _tc_heads(g, w1, b1, w2, b2, nw1, nb1, nw2, nb2, cw1, cb1, cw2, cb2):
    return pl.pallas_call(
        _heads_body,
        out_shape=[
            jax.ShapeDtypeStruct((NUM_GRAPHS, D), jnp.float32),
            jax.ShapeDtypeStruct((NUM_GRAPHS, 1), jnp.float32),
            jax.ShapeDtypeStruct((NUM_GRAPHS, 10), jnp.float32),
        ],
    )(g, w1, b1, w2, b2, nw1, nb1, nw2, nb2, cw1, cb1, cw2, cb2)


@functools.partial(
    pl.kernel,
    out_type=jax.ShapeDtypeStruct((EP,), jnp.float32),
    mesh=_MESH,
    compiler_params=_SC_PARAMS,
    scratch_types=[
        pltpu.VMEM((C,), jnp.int32),
        pltpu.VMEM((C,), jnp.int32),
        pltpu.VMEM((C, 16), jnp.float32),
        pltpu.VMEM((C, 16), jnp.float32),
        pltpu.VMEM((C,), jnp.float32),
        pltpu.SemaphoreType.DMA,
        pltpu.SemaphoreType.DMA,
    ],
)
def _sc_dsq(pos_hbm, src_hbm, dst_hbm, out_hbm, si, di, pa, pb, dbuf, s1, s2):
    wid = lax.axis_index("s") * 2 + lax.axis_index("c")
    lane = lax.broadcasted_iota(jnp.int32, (16,), 0)

    def chunk(ci, carry):
        base = wid * EW + ci * C
        pltpu.sync_copy(src_hbm.at[pl.ds(base, C)], si)
        pltpu.sync_copy(dst_hbm.at[pl.ds(base, C)], di)
        ca = pltpu.async_copy(pos_hbm.at[si], pa, s1)
        cb = pltpu.async_copy(pos_hbm.at[di], pb, s2)
        ca.wait()
        cb.wait()

        def grp(g, carry2):
            acc = jnp.zeros((16,), jnp.float32)
            for t in range(16):
                i = g * 16 + t
                dv = pa[i, :] - pb[i, :] + 1e-8
                s = jnp.sum(dv * dv)
                acc = jnp.where(lane == t, s, acc)
            dbuf[pl.ds(g * 16, 16)] = acc
            return carry2

        lax.fori_loop(0, C // 16, grp, 0)
        pltpu.sync_copy(dbuf, out_hbm.at[pl.ds(base, C)])
        return carry

    lax.fori_loop(0, EW // C, chunk, 0)


@functools.partial(
    pl.kernel,
    out_type=jax.ShapeDtypeStruct((2, NP, ACCW), jnp.float32),
    mesh=_MESH,
    compiler_params=_SC_PARAMS,
    scratch_types=[
        pltpu.VMEM((C,), jnp.int32),
        pltpu.VMEM((C,), jnp.int32),
        pltpu.VMEM((C, MSG), jnp.float32),
        pltpu.VMEM((C, MSG), jnp.float32),
        pltpu.VMEM((C, MSG), jnp.float32),
        pltpu.VMEM((C, MSG), jnp.float32),
        pltpu.VMEM((C, ACCW), jnp.float32),
        pltpu.VMEM_SHARED((NP, ACCW), jnp.float32),
        pltpu.SemaphoreType.DMA,
        pltpu.SemaphoreType.DMA,
        pltpu.SemaphoreType.DMA,
        pltpu.SemaphoreType.DMA,
    ],
)
def _sc_edge(qh, kh, vh, eh, src, dst, out,
             si, di, qb, kb, vb, eb, rb, acc, s1, s2, s3, s4):
    cid = lax.axis_index("c")
    sid = lax.axis_index("s")
    wid = sid * 2 + cid
    lane = lax.broadcasted_iota(jnp.int32, (16,), 0)
    zero16 = jnp.zeros((16,), jnp.float32)

    def zrow(i, carry):
        for j in range(ACCW // 16):
            rb[i, pl.ds(j * 16, 16)] = zero16
        return carry

    lax.fori_loop(0, C, zrow, 0)
    for j in range(5):
        pltpu.sync_copy(rb, acc.at[pl.ds(sid * 640 + j * C, C)])
    plsc.subcore_barrier()

    def chunk(ci, carry):
        base = wid * EW + ci * C
        pltpu.sync_copy(src.at[pl.ds(base, C)], si)
        pltpu.sync_copy(dst.at[pl.ds(base, C)], di)
        cq = pltpu.async_copy(qh.at[di], qb, s1)
        ck = pltpu.async_copy(kh.at[si], kb, s2)
        cv = pltpu.async_copy(vh.at[si], vb, s3)
        ce = pltpu.async_copy(eh.at[pl.ds(base, C)], eb, s4)
        cq.wait()
        ck.wait()
        cv.wait()
        ce.wait()

        def grp(g, carry2):
            dots = [zero16, zero16, zero16, zero16]
            for t in range(16):
                i = g * 16 + t
                for hh in range(H):
                    qv = qb[i, pl.ds(hh * DH, DH)]
                    kv = kb[i, pl.ds(hh * DH, DH)] + eb[i, pl.ds(hh * DH, DH)]
                    s = jnp.sum(qv * kv) * 0.25
                    dots[hh] = jnp.where(lane == t, s, dots[hh])
            ps = [jnp.exp(dv) for dv in dots]
            for t in range(16):
                i = g * 16 + t
                den = zero16
                for hh in range(H):
                    pv = ps[hh][t]
                    wv = vb[i, pl.ds(hh * DH, DH)] + eb[i, pl.ds(hh * DH, DH)]
                    rb[i, pl.ds(hh * DH, DH)] = pv * wv
                    den = jnp.where(lane == hh, pv, den)
                rb[i, pl.ds(MSG, 16)] = den
            return carry2

        lax.fori_loop(0, C // 16, grp, 0)
        pltpu.sync_copy(rb, acc.at[di], add=True)
        return carry

    lax.fori_loop(0, EW // C, chunk, 0)
    plsc.subcore_barrier()
    for j in range(5):
        r0 = sid * 640 + j * C
        pltpu.sync_copy(acc.at[pl.ds(r0, C)], out.at[cid, pl.ds(r0, C)])


def kernel(x, pos, edge_index, batch, params):
    f32 = jnp.float32
    x2 = jnp.pad(x.astype(jnp.int32), (0, NP - N)).reshape(NP, 1)
    pos3 = jnp.pad(pos, ((0, NP - N), (0, 0)))
    pos16 = jnp.pad(pos, ((0, NP - N), (0, 13)))
    src = jnp.pad(edge_index[0], (0, EP - E), constant_values=N)
    dst = jnp.pad(edge_index[1], (0, EP - E), constant_values=N)
    batch2 = batch.astype(jnp.int32).reshape(N, 1)
    p = params

    h = _tc_h0(x2, pos3, p["atom_emb"],
               p["in_w"], p["in_b"].reshape(1, D))

    dsq = _sc_dsq(pos16, src, dst)
    we_all = jnp.stack([jnp.pad(lp["we"], ((0, 64 - NUM_RADIAL), (0, 0)))
                        for lp in p["layers"]])
    e_all = _tc_rbfe(dsq.reshape(EP, 1), we_all)

    for l in range(L):
        lp = p["layers"][l]
        qh, kh, vh = _tc_qkv(h, lp["wq"], lp["wk"], lp["wv"])
        acc = _sc_edge(qh, kh, vh, e_all[l], src, dst)
        (f1w, f1b), (f2w, f2b) = lp["fc"]
        h = _tc_post(h, acc, lp["wo"], lp["bo"].reshape(1, D),
                     lp["ln_g"].reshape(1, D), lp["ln_b"].reshape(1, D),
                     f1w, f1b.reshape(1, D), f2w, f2b.reshape(1, D))

    g = _tc_pool(h, batch2)
    (w1, b1), (w2, b2) = p["fcs"]
    enc, na, cp = _tc_heads(
        g, w1, b1.reshape(1, D), w2, b2.reshape(1, D),
        p["na_w1"], p["na_b1"].reshape(1, 32),
        p["na_w2"], p["na_b2"].reshape(1, 1),
        p["cp_w1"], p["cp_b1"].reshape(1, 32),
        p["cp_w2"], p["cp_b2"].reshape(1, 10))
    return enc, na, cp


# speedup vs baseline: 39.3062x; 1.3367x over previous
import functools

import jax
import jax.numpy as jnp
from jax import lax
from jax.experimental import pallas as pl
from jax.experimental.pallas import tpu as pltpu
from jax.experimental.pallas import tpu_sc as plsc

N = 10000
NP = 10240
E = 160000
EP = 163840
EW = EP // 32
C = 128
D = 128
MSG = 64
H = 4
DH = 16
L = 4
NUM_RADIAL = 50
CUTOFF = 6.0
NUM_GRAPHS = 100
ATOM_TYPES = 5
ACCW = 80

_MESH = plsc.VectorSubcoreMesh(core_axis_name="c", subcore_axis_name="s")
_SC_PARAMS = pltpu.CompilerParams(needs_layout_passes=False,
                                  use_tc_tiling_on_sc=False)


def _h0_body(x_ref, pos_ref, emb_ref, w_ref, b_ref, o_ref):
    xb = x_ref[...]
    oh = (xb == lax.broadcasted_iota(jnp.int32, (1, ATOM_TYPES), 1))
    ex = jnp.dot(oh.astype(jnp.float32), emb_ref[...],
                 preferred_element_type=jnp.float32)
    feat = jnp.concatenate([ex, pos_ref[...]], axis=1)
    o_ref[...] = jnp.dot(feat, w_ref[...],
                         preferred_element_type=jnp.float32) + b_ref[...]


def _tc_h0(x2, pos3, emb, w, b):
    blk = 2560
    return pl.pallas_call(
        _h0_body,
        grid=(NP // blk,),
        in_specs=[
            pl.BlockSpec((blk, 1), lambda i: (i, 0)),
            pl.BlockSpec((blk, 3), lambda i: (i, 0)),
            pl.BlockSpec((ATOM_TYPES, ATOM_TYPES), lambda i: (0, 0)),
            pl.BlockSpec((ATOM_TYPES + 3, D), lambda i: (0, 0)),
            pl.BlockSpec((1, D), lambda i: (0, 0)),
        ],
        out_specs=pl.BlockSpec((blk, D), lambda i: (i, 0)),
        out_shape=jax.ShapeDtypeStruct((NP, D), jnp.float32),
    )(x2, pos3, emb, w, b)


def _rbfe_body(dsq_ref, we_ref, o_ref):
    d = jnp.sqrt(dsq_ref[...])
    j = lax.broadcasted_iota(jnp.int32, (1, 64), 1).astype(jnp.float32)
    centers = j * (CUTOFF / (NUM_RADIAL - 1))
    gamma = CUTOFF / NUM_RADIAL
    rbf = jnp.exp(-((d - centers) ** 2) * (1.0 / (2.0 * gamma * gamma)))
    for l in range(L):
        o_ref[l, :, :] = jnp.dot(rbf, we_ref[l],
                                 preferred_element_type=jnp.float32)


def _tc_rbfe(dsq2, we_all):
    blk = 2048
    return pl.pallas_call(
        _rbfe_body,
        grid=(EP // blk,),
        in_specs=[
            pl.BlockSpec((blk, 1), lambda i: (i, 0)),
            pl.BlockSpec((L, 64, MSG), lambda i: (0, 0, 0)),
        ],
        out_specs=pl.BlockSpec((L, blk, MSG), lambda i: (0, i, 0)),
        out_shape=jax.ShapeDtypeStruct((L, EP, MSG), jnp.float32),
    )(dsq2, we_all)


def _qkv_body(h_ref, wq_ref, wk_ref, wv_ref, q_ref, kv_ref):
    hb = h_ref[...]
    q_ref[...] = jnp.dot(hb, wq_ref[...], preferred_element_type=jnp.float32)
    kk = jnp.dot(hb, wk_ref[...], preferred_element_type=jnp.float32)
    vv = jnp.dot(hb, wv_ref[...], preferred_element_type=jnp.float32)
    kv_ref[...] = jnp.concatenate([kk, vv], axis=1)


def _tc_qkv(h, wq, wk, wv):
    blk = 2560
    w_spec = pl.BlockSpec((D, MSG), lambda i: (0, 0))
    return pl.pallas_call(
        _qkv_body,
        grid=(NP // blk,),
        in_specs=[pl.BlockSpec((blk, D), lambda i: (i, 0)),
                  w_spec, w_spec, w_spec],
        out_specs=[pl.BlockSpec((blk, MSG), lambda i: (i, 0)),
                   pl.BlockSpec((blk, 2 * MSG), lambda i: (i, 0))],
        out_shape=[jax.ShapeDtypeStruct((NP, MSG), jnp.float32),
                   jax.ShapeDtypeStruct((NP, 2 * MSG), jnp.float32)],
    )(h, wq, wk, wv)


def _post_body(h_ref, acc_ref, wo_ref, bo_ref, g_ref, b_ref,
               f1w_ref, f1b_ref, f2w_ref, f2b_ref, o_ref):
    a = acc_ref[0] + acc_ref[1]
    parts = []
    for hh in range(H):
        num = a[:, hh * DH:(hh + 1) * DH]
        den = a[:, MSG + hh:MSG + hh + 1]
        parts.append(num / (den + 1e-30))
    msg = jnp.concatenate(parts, axis=1)
    u = jnp.dot(msg, wo_ref[...], preferred_element_type=jnp.float32)
    x1 = h_ref[...] + jax.nn.gelu(u + bo_ref[...])
    mu = jnp.mean(x1, axis=1, keepdims=True)
    xc = x1 - mu
    var = jnp.mean(xc * xc, axis=1, keepdims=True)
    x1 = xc / jnp.sqrt(var + 1e-5) * g_ref[...] + b_ref[...]
    x1 = x1 + jax.nn.gelu(
        jnp.dot(x1, f1w_ref[...], preferred_element_type=jnp.float32)
        + f1b_ref[...])
    x1 = x1 + jax.nn.gelu(
        jnp.dot(x1, f2w_ref[...], preferred_element_type=jnp.float32)
        + f2b_ref[...])
    o_ref[...] = x1


def _tc_post(h, acc, wo, bo, ln_g, ln_b, f1w, f1b, f2w, f2b):
    blk = 2560
    vec = lambda: pl.BlockSpec((1, D), lambda i: (0, 0))
    mat = lambda: pl.BlockSpec((D, D), lambda i: (0, 0))
    return pl.pallas_call(
        _post_body,
        grid=(NP // blk,),
        in_specs=[
            pl.BlockSpec((blk, D), lambda i: (i, 0)),
            pl.BlockSpec((2, blk, ACCW), lambda i: (0, i, 0)),
            pl.BlockSpec((MSG, D), lambda i: (0, 0)), vec(),
            vec(), vec(),
            mat(), vec(), mat(), vec(),
        ],
        out_specs=pl.BlockSpec((blk, D), lambda i: (i, 0)),
        out_shape=jax.ShapeDtypeStruct((NP, D), jnp.float32),
    )(h, acc, wo, bo, ln_g, ln_b, f1w, f1b, f2w, f2b)


def _pool_body(h_ref, b_ref, o_ref):
    i = pl.program_id(0)
    oh = (b_ref[...] == lax.broadcasted_iota(jnp.int32, (1, 128), 1))
    part = lax.dot_general(oh.astype(jnp.float32), h_ref[...],
                           (((0,), (0,)), ((), ())),
                           preferred_element_type=jnp.float32)

    @pl.when(i == 0)
    def _():
        o_ref[...] = jnp.zeros_like(o_ref)

    o_ref[...] += part


def _tc_pool(h, batch2):
    blk = 2000
    return pl.pallas_call(
        _pool_body,
        grid=(N // blk,),
        in_specs=[
            pl.BlockSpec((blk, D), lambda i: (i, 0)),
            pl.BlockSpec((blk, 1), lambda i: (i, 0)),
        ],
        out_specs=pl.BlockSpec((128, D), lambda i: (0, 0)),
        out_shape=jax.ShapeDtypeStruct((128, D), jnp.float32),
    )(h, batch2)


def _heads_body(g_ref, w1_ref, b1_ref, w2_ref, b2_ref,
                nw1_ref, nb1_ref, nw2_ref, nb2_ref,
                cw1_ref, cb1_ref, cw2_ref, cb2_ref,
                enc_ref, na_ref, cp_ref):
    g = g_ref[...]
    g = jax.nn.gelu(jnp.dot(g, w1_ref[...],
                            preferred_element_type=jnp.float32) + b1_ref[...])
    g = jax.nn.gelu(jnp.dot(g, w2_ref[...],
                            preferred_element_type=jnp.float32) + b2_ref[...])
    enc_ref[...] = g[:NUM_GRAPHS, :]
    t = jax.nn.gelu(jnp.dot(g, nw1_ref[...],
                            preferred_element_type=jnp.float32) + nb1_ref[...])
    na = jnp.dot(t, nw2_ref[...], preferred_element_type=jnp.float32) \
        + nb2_ref[...]
    na_ref[...] = na[:NUM_GRAPHS, :]
    u = jax.nn.gelu(jnp.dot(g, cw1_ref[...],
                            preferred_element_type=jnp.float32) + cb1_ref[...])
    cp = jnp.dot(u, cw2_ref[...], preferred_element_type=jnp.float32) \
        + cb2_ref[...]
    cp_ref[...] = cp[:NUM_GRAPHS, :]


def _tc_heads(g, w1, b1, w2, b2, nw1, nb1, nw2, nb2, cw1, cb1, cw2, cb2):
    return pl.pallas_call(
        _heads_body,
        out_shape=[
            jax.ShapeDtypeStruct((NUM_GRAPHS, D), jnp.float32),
            jax.ShapeDtypeStruct((NUM_GRAPHS, 1), jnp.float32),
            jax.ShapeDtypeStruct((NUM_GRAPHS, 10), jnp.float32),
        ],
    )(g, w1, b1, w2, b2, nw1, nb1, nw2, nb2, cw1, cb1, cw2, cb2)


@functools.partial(
    pl.kernel,
    out_type=jax.ShapeDtypeStruct((EP,), jnp.float32),
    mesh=_MESH,
    compiler_params=_SC_PARAMS,
    scratch_types=[
        pltpu.VMEM((C,), jnp.int32),
        pltpu.VMEM((C,), jnp.int32),
        pltpu.VMEM((C, 16), jnp.float32),
        pltpu.VMEM((C, 16), jnp.float32),
        pltpu.VMEM((C,), jnp.float32),
        pltpu.SemaphoreType.DMA,
        pltpu.SemaphoreType.DMA,
    ],
)
def _sc_dsq(pos_hbm, src_hbm, dst_hbm, out_hbm, si, di, pa, pb, dbuf, s1, s2):
    wid = lax.axis_index("s") * 2 + lax.axis_index("c")
    lane = lax.broadcasted_iota(jnp.int32, (16,), 0)

    def chunk(ci, carry):
        base = wid * EW + ci * C
        pltpu.sync_copy(src_hbm.at[pl.ds(base, C)], si)
        pltpu.sync_copy(dst_hbm.at[pl.ds(base, C)], di)
        ca = pltpu.async_copy(pos_hbm.at[si], pa, s1)
        cb = pltpu.async_copy(pos_hbm.at[di], pb, s2)
        ca.wait()
        cb.wait()

        def grp(g, carry2):
            acc = jnp.zeros((16,), jnp.float32)
            for t in range(16):
                i = g * 16 + t
                dv = pa[i, :] - pb[i, :] + 1e-8
                s = jnp.sum(dv * dv)
                acc = jnp.where(lane == t, s, acc)
            dbuf[pl.ds(g * 16, 16)] = acc
            return carry2

        lax.fori_loop(0, C // 16, grp, 0)
        pltpu.sync_copy(dbuf, out_hbm.at[pl.ds(base, C)])
        return carry

    lax.fori_loop(0, EW // C, chunk, 0)


NCH = EW // C


@functools.partial(
    pl.kernel,
    out_type=jax.ShapeDtypeStruct((2, NP, ACCW), jnp.float32),
    mesh=_MESH,
    compiler_params=_SC_PARAMS,
    scratch_types=[
        pltpu.VMEM((NCH, C), jnp.int32),
        pltpu.VMEM((NCH, C), jnp.int32),
        pltpu.VMEM((2, C, MSG), jnp.float32),
        pltpu.VMEM((2, C, 2 * MSG), jnp.float32),
        pltpu.VMEM((C, MSG), jnp.float32),
        pltpu.VMEM((C, ACCW), jnp.float32),
        pltpu.VMEM((C,), jnp.int32),
        pltpu.VMEM_SHARED((NP, ACCW), jnp.float32),
        pltpu.SemaphoreType.DMA,
        pltpu.SemaphoreType.DMA,
        pltpu.SemaphoreType.DMA,
        pltpu.SemaphoreType.DMA,
    ],
)
def _sc_edge(qh, kvh, eh, src2, dst2, out,
             si, di, qb, kvb, eb, rb, dsc, acc, sq0, sq1, sk0, sk1):
    cid = lax.axis_index("c")
    sid = lax.axis_index("s")
    wid = sid * 2 + cid
    lane = lax.broadcasted_iota(jnp.int32, (16,), 0)
    zero16 = jnp.zeros((16,), jnp.float32)
    qsems = [sq0, sq1]
    ksems = [sk0, sk1]

    def zrow(i, carry):
        for j in range(ACCW // 16):
            rb[i, pl.ds(j * 16, 16)] = zero16
        return carry

    lax.fori_loop(0, C, zrow, 0)
    for j in range(5):
        pltpu.sync_copy(rb, acc.at[pl.ds(sid * 640 + j * C, C)])

    pltpu.sync_copy(src2.at[pl.ds(wid * NCH, NCH)], si)
    pltpu.sync_copy(dst2.at[pl.ds(wid * NCH, NCH)], di)
    plsc.subcore_barrier()

    def issue(ci, b):
        pltpu.async_copy(qh.at[di.at[ci]], qb.at[b], qsems[b])
        pltpu.async_copy(kvh.at[si.at[ci]], kvb.at[b], ksems[b])

    def drain(ci, b):
        pltpu.sync_copy(eh.at[pl.ds((wid * NCH + ci) * C, C)], eb)
        pltpu.make_async_copy(qh.at[di.at[ci]], qb.at[b], qsems[b]).wait()
        pltpu.make_async_copy(kvh.at[si.at[ci]], kvb.at[b], ksems[b]).wait()

    def compute(b):
        def grp(g, carry2):
            dots = [zero16, zero16, zero16, zero16]
            for t in range(16):
                i = g * 16 + t
                for hh in range(H):
                    qv = qb[b, i, pl.ds(hh * DH, DH)]
                    kv = kvb[b, i, pl.ds(hh * DH, DH)] \
                        + eb[i, pl.ds(hh * DH, DH)]
                    s = jnp.sum(qv * kv) * 0.25
                    dots[hh] = jnp.where(lane == t, s, dots[hh])
            ps = [jnp.exp(dv) for dv in dots]
            for t in range(16):
                i = g * 16 + t
                den = zero16
                for hh in range(H):
                    pv = ps[hh][t]
                    wv = kvb[b, i, pl.ds(MSG + hh * DH, DH)] \
                        + eb[i, pl.ds(hh * DH, DH)]
                    rb[i, pl.ds(hh * DH, DH)] = pv * wv
                    den = jnp.where(lane == hh, pv, den)
                rb[i, pl.ds(MSG, 16)] = den
            return carry2

        lax.fori_loop(0, C // 16, grp, 0)

    issue(0, 0)

    def pair(c2, carry):
        for b in range(2):
            ci = c2 + b

            @pl.when(ci + 1 < NCH)
            def _():
                issue(ci + 1, 1 - b)

            for k in range(C // 16):
                dsc[pl.ds(k * 16, 16)] = di[ci, pl.ds(k * 16, 16)]
            drain(ci, b)
            compute(b)
            pltpu.sync_copy(rb, acc.at[dsc], add=True)
        return carry

    lax.fori_loop(0, NCH // 2, lambda j, c: pair(j * 2, c), 0)
    plsc.subcore_barrier()
    for j in range(5):
        r0 = sid * 640 + j * C
        pltpu.sync_copy(acc.at[pl.ds(r0, C)], out.at[cid, pl.ds(r0, C)])


def kernel(x, pos, edge_index, batch, params):
    f32 = jnp.float32
    x2 = jnp.pad(x.astype(jnp.int32), (0, NP - N)).reshape(NP, 1)
    pos3 = jnp.pad(pos, ((0, NP - N), (0, 0)))
    pos16 = jnp.pad(pos, ((0, NP - N), (0, 13)))
    src = jnp.pad(edge_index[0], (0, EP - E), constant_values=N)
    dst = jnp.pad(edge_index[1], (0, EP - E), constant_values=N)
    src2 = src.reshape(EP // C, C)
    dst2 = dst.reshape(EP // C, C)
    batch2 = batch.astype(jnp.int32).reshape(N, 1)
    p = params

    h = _tc_h0(x2, pos3, p["atom_emb"],
               p["in_w"], p["in_b"].reshape(1, D))

    dsq = _sc_dsq(pos16, src, dst)
    we_all = jnp.stack([jnp.pad(lp["we"], ((0, 64 - NUM_RADIAL), (0, 0)))
                        for lp in p["layers"]])
    e_all = _tc_rbfe(dsq.reshape(EP, 1), we_all)

    for l in range(L):
        lp = p["layers"][l]
        qh, kvh = _tc_qkv(h, lp["wq"], lp["wk"], lp["wv"])
        acc = _sc_edge(qh, kvh, e_all[l], src2, dst2)
        (f1w, f1b), (f2w, f2b) = lp["fc"]
        h = _tc_post(h, acc, lp["wo"], lp["bo"].reshape(1, D),
                     lp["ln_g"].reshape(1, D), lp["ln_b"].reshape(1, D),
                     f1w, f1b.reshape(1, D), f2w, f2b.reshape(1, D))

    g = _tc_pool(h, batch2)
    (w1, b1), (w2, b2) = p["fcs"]
    enc, na, cp = _tc_heads(
        g, w1, b1.reshape(1, D), w2, b2.reshape(1, D),
        p["na_w1"], p["na_b1"].reshape(1, 32),
        p["na_w2"], p["na_b2"].reshape(1, 1),
        p["cp_w1"], p["cp_b1"].reshape(1, 32),
        p["cp_w2"], p["cp_b2"].reshape(1, 10))
    return enc, na, cp
